# baseline (speedup 1.0000x reference)
import functools

import jax
import jax.numpy as jnp
from jax import lax
from jax.experimental import pallas as pl
from jax.experimental.pallas import tpu as pltpu
from jax.experimental.pallas import tpu_sc as plsc

f32 = jnp.float32
i32 = jnp.int32

C = 128
PH = 128
BN = 2000
BE = 2048
NLAYER = 9
NRANGE = 32
RSZ = 320
CAP = 6144
CAPC = CAP // C


def _emlp_body(ea_ref, w1_ref, b1_ref, w2_ref, b2_ref, rec_ref, rt_ref, *,
               be):
    h = jnp.maximum(
        jnp.dot(ea_ref[...], w1_ref[...], preferred_element_type=f32)
        + b1_ref[...], 0.0)
    logit = jnp.dot(h, w2_ref[...], preferred_element_type=f32) + b2_ref[...]
    s = 1.0 / (1.0 + jnp.exp(-logit))
    col = lax.broadcasted_iota(i32, (be, 16), 1)
    rec = jnp.where(col < NLAYER, s,
                    jnp.where(col == NLAYER, 1.0, 0.0))
    rec_ref[...] = rec
    rt_ref[...] = rec.T


def _edge_mlp(eap, w1cat, b1cat, w2bd, b2cat):
    tot = eap.shape[0]
    grid = tot // BE
    return pl.pallas_call(
        functools.partial(_emlp_body, be=BE),
        grid=(grid,),
        in_specs=[
            pl.BlockSpec((BE, 16), lambda i: (i, 0)),
            pl.BlockSpec((16, 288), lambda i: (0, 0)),
            pl.BlockSpec((1, 288), lambda i: (0, 0)),
            pl.BlockSpec((288, 16), lambda i: (0, 0)),
            pl.BlockSpec((1, 16), lambda i: (0, 0)),
        ],
        out_specs=[
            pl.BlockSpec((BE, 16), lambda i: (i, 0)),
            pl.BlockSpec((16, BE), lambda i: (0, i)),
        ],
        out_shape=[
            jax.ShapeDtypeStruct((tot, 16), f32),
            jax.ShapeDtypeStruct((16, tot), f32),
        ],
    )(eap, w1cat, b1cat, w2bd, b2cat)


def _dinv_body(d_ref, o_ref):
    o_ref[...] = lax.rsqrt(d_ref[...] + 1.0)


def _dinv(degp, n):
    return pl.pallas_call(
        _dinv_body,
        grid=(n // BN,),
        in_specs=[pl.BlockSpec((BN, 16), lambda i: (i, 0))],
        out_specs=pl.BlockSpec((BN, 16), lambda i: (i, 0)),
        out_shape=jax.ShapeDtypeStruct((n, 16), f32),
    )(degp)


def _mm_body(*refs, l, cout, colsplit, gnorm, ntot):
    if gnorm:
        h_ref, sums_ref, gw_ref, gb_ref, gms_ref, lin_ref, dinv_ref, o_ref = refs
        mean = sums_ref[0:1, :] / ntot
        ez2 = sums_ref[1:2, :] / ntot
        ms = gms_ref[...]
        var = ez2 - (2.0 * ms - ms * ms) * mean * mean
        a = gw_ref[...] * lax.rsqrt(var + 1e-5)
        b = gb_ref[...] - a * mean * ms
        u = refs[0][...] * a + b
    else:
        h_ref, lin_ref, dinv_ref, o_ref = refs
        u = h_ref[...]
    xw = jnp.dot(u, lin_ref[...], preferred_element_type=f32)
    y = xw * dinv_ref[...][:, l:l + 1]
    nb = y.shape[0]
    if colsplit:
        r = cout - PH
        o_ref[0] = y[:, :PH]
        o_ref[1, :, :r] = y[:, PH:]
        if r < PH:
            o_ref[1, :, r:] = jnp.zeros((nb, PH - r), f32)
    else:
        o_ref[:, :cout] = y
        if cout < PH:
            o_ref[:, cout:] = jnp.zeros((nb, PH - cout), f32)


def _mm(h, sums, gn, lin, dinv, l):
    n, cin = h.shape
    cout = lin.shape[1]
    colsplit = cout > PH
    gnorm = sums is not None
    body = functools.partial(_mm_body, l=l, cout=cout, colsplit=colsplit,
                             gnorm=gnorm, ntot=float(n))
    in_specs = [pl.BlockSpec((BN, cin), lambda i: (i, 0))]
    args = [h]
    if gnorm:
        in_specs += [
            pl.BlockSpec((2, cin), lambda i: (0, 0)),
            pl.BlockSpec((1, cin), lambda i: (0, 0)),
            pl.BlockSpec((1, cin), lambda i: (0, 0)),
            pl.BlockSpec((1, cin), lambda i: (0, 0)),
        ]
        args += [sums, gn["weight"][None], gn["bias"][None],
                 gn["mean_scale"][None]]
    in_specs += [
        pl.BlockSpec((cin, cout), lambda i: (0, 0)),
        pl.BlockSpec((BN, 16), lambda i: (i, 0)),
    ]
    args += [lin, dinv]
    if colsplit:
        out_specs = pl.BlockSpec((2, BN, PH), lambda i: (0, i, 0))
        out_shape = jax.ShapeDtypeStruct((2, n, PH), f32)
    else:
        out_specs = pl.BlockSpec((BN, PH), lambda i: (i, 0))
        out_shape = jax.ShapeDtypeStruct((n, PH), f32)
    return pl.pallas_call(
        body,
        grid=(n // BN,),
        in_specs=in_specs,
        out_specs=out_specs,
        out_shape=out_shape,
    )(*args)


def _post_body(agg_ref, y_ref, dinv_ref, bias_ref, h_ref, sums_ref, *,
               l, cout, colsplit, relu):
    i = pl.program_id(0)
    if colsplit:
        a0 = agg_ref[0] + y_ref[0]
        a1 = agg_ref[1] + y_ref[1]
        z = jnp.concatenate([a0, a1[:, :cout - PH]], axis=1)
    else:
        z = (agg_ref[...] + y_ref[...])[:, :cout]
    z = z * dinv_ref[...][:, l:l + 1] + bias_ref[...]
    if relu:
        z = jnp.maximum(z, 0.0)
    h_ref[...] = z
    ps = jnp.concatenate([jnp.sum(z, axis=0, keepdims=True),
                          jnp.sum(z * z, axis=0, keepdims=True)], axis=0)

    @pl.when(i == 0)
    def _():
        sums_ref[...] = ps

    @pl.when(i > 0)
    def _():
        sums_ref[...] += ps


def _post(agg, y, dinv, bias, l, cout):
    colsplit = y.ndim == 3
    n = y.shape[1] if colsplit else y.shape[0]
    body = functools.partial(_post_body, l=l, cout=cout, colsplit=colsplit,
                             relu=True)
    if colsplit:
        agg_spec = pl.BlockSpec((2, BN, PH), lambda i: (0, i, 0))
        y_spec = pl.BlockSpec((2, BN, PH), lambda i: (0, i, 0))
    else:
        agg_spec = pl.BlockSpec((BN, PH), lambda i: (i, 0))
        y_spec = pl.BlockSpec((BN, PH), lambda i: (i, 0))
    return pl.pallas_call(
        body,
        grid=(n // BN,),
        in_specs=[
            agg_spec,
            y_spec,
            pl.BlockSpec((BN, 16), lambda i: (i, 0)),
            pl.BlockSpec((1, cout), lambda i: (0, 0)),
        ],
        out_specs=[
            pl.BlockSpec((BN, cout), lambda i: (i, 0)),
            pl.BlockSpec((2, cout), lambda i: (0, 0)),
        ],
        out_shape=[
            jax.ShapeDtypeStruct((n, cout), f32),
            jax.ShapeDtypeStruct((2, cout), f32),
        ],
    )(agg, y, dinv, bias)


def _final_body(agg_ref, y_ref, dinv_ref, bias_ref, mu_ref, ls_ref, *, l):
    z = agg_ref[...] + y_ref[...]
    z = z * dinv_ref[...][:, l:l + 1] + bias_ref[...]
    mu_ref[...] = z[:, :64]
    ls_ref[...] = z[:, 64:]


def _final(agg, y, dinv, bias):
    n = y.shape[0]
    body = functools.partial(_final_body, l=NLAYER)
    return pl.pallas_call(
        body,
        grid=(n // BN,),
        in_specs=[
            pl.BlockSpec((BN, PH), lambda i: (i, 0)),
            pl.BlockSpec((BN, PH), lambda i: (i, 0)),
            pl.BlockSpec((BN, 16), lambda i: (i, 0)),
            pl.BlockSpec((1, 128), lambda i: (0, 0)),
        ],
        out_specs=[
            pl.BlockSpec((BN, 64), lambda i: (i, 0)),
            pl.BlockSpec((BN, 64), lambda i: (i, 0)),
        ],
        out_shape=[
            jax.ShapeDtypeStruct((n, 64), f32),
            jax.ShapeDtypeStruct((n, 64), f32),
        ],
    )(agg, y, dinv, bias)


_MESH = dict(core_axis_name="c", subcore_axis_name="s")


def _build_deg(tot, npad):
    totc = tot // C
    accr = RSZ + 8
    mesh = plsc.VectorSubcoreMesh(**_MESH)

    @functools.partial(
        pl.kernel, mesh=mesh,
        out_type=jax.ShapeDtypeStruct((npad, 16), f32),
        scratch_types=[
            pltpu.VMEM((C, 16), f32),
            pltpu.VMEM((C, 16), f32),
            pltpu.VMEM((C,), i32),
            pltpu.VMEM((C,), i32),
            pltpu.SemaphoreType.DMA,
            pltpu.SemaphoreType.DMA,
            pltpu.SemaphoreType.DMA,
            pltpu.SemaphoreType.DMA,
            pltpu.VMEM((accr, 16), f32),
        ])
    def deg_kernel(rec, dlr, out, reca, recb, dla, dlb, sra, srb, sda, sdb,
                   acc):
        h = lax.axis_index("c")
        s = lax.axis_index("s")
        r = h * 16 + s
        cb = r * CAPC

        def zrow(i, carry):
            acc[i, :] = jnp.zeros((16,), f32)
            return carry
        lax.fori_loop(0, accr, zrow, 0, unroll=4)

        def rows(gg):
            return pl.ds(pl.multiple_of((cb + gg) * C, C), C)

        def issue(gg, rb, db, sr, sd):
            pltpu.async_copy(rec.at[rows(gg)], rb, sr)
            pltpu.async_copy(dlr.at[cb + gg], db, sd)

        def wait(rb, db, sr, sd):
            pltpu.make_async_copy(rec.at[rows(0)], rb, sr).wait()
            pltpu.make_async_copy(dlr.at[0], db, sd).wait()

        def accum(rb, db):
            def one(q, carry):
                dlv = db[pl.ds(q * 16, 16)]
                for k in range(16):
                    e1 = q * 16 + k
                    dl = dlv[k]
                    acc[dl, :] = acc[dl, :] + rb[e1, :]
                return carry
            lax.fori_loop(0, C // 16, one, 0)

        issue(0, reca, dla, sra, sda)
        issue(1, recb, dlb, srb, sdb)

        def body(gp, carry):
            g0 = 2 * gp
            wait(reca, dla, sra, sda)
            accum(reca, dla)
            issue(jnp.minimum(g0 + 2, CAPC - 1), reca, dla, sra, sda)
            wait(recb, dlb, srb, sdb)
            accum(recb, dlb)
            issue(jnp.minimum(g0 + 3, CAPC - 1), recb, dlb, srb, sdb)
            return carry
        lax.fori_loop(0, CAPC // 2, body, 0)
        wait(reca, dla, sra, sda)
        wait(recb, dlb, srb, sdb)
        pltpu.sync_copy(acc.at[pl.ds(0, RSZ)], out.at[pl.ds(r * RSZ, RSZ)])

    return deg_kernel


def _build_prop(tot, npad, colsplit):
    nchunk = CAPC
    accr = RSZ + 8
    nj = PH // 16
    mesh = plsc.VectorSubcoreMesh(**_MESH)

    @functools.partial(
        pl.kernel, mesh=mesh,
        out_type=(jax.ShapeDtypeStruct((2, npad, PH), f32) if colsplit
                  else jax.ShapeDtypeStruct((npad, PH), f32)),
        scratch_types=(
            [pltpu.VMEM((C, PH), f32)] * 4
            + [pltpu.VMEM((C,), f32)] * 4
            + [pltpu.VMEM((C,), i32)] * 8
            + [pltpu.SemaphoreType.DMA] * 8
            + [pltpu.VMEM((accr, PH), f32)]
        ))
    def prop_kernel(y2f, srcp2, dlr, gater, out,
                    row0, row1, row2, row3, gt0, gt1, gt2, gt3,
                    ix0, ix1, ix2, ix3, dl0, dl1, dl2, dl3,
                    sg0, sg1, sg2, sg3, sm0, sm1, sm2, sm3, acc):
        h = lax.axis_index("c")
        s = lax.axis_index("s")
        hsel = h if colsplit else 0
        rows_ = [row0, row1, row2, row3]
        gts = [gt0, gt1, gt2, gt3]
        ixs = [ix0, ix1, ix2, ix3]
        dls = [dl0, dl1, dl2, dl3]
        sgs = [sg0, sg1, sg2, sg3]
        sms = [sm0, sm1, sm2, sm3]

        def one_range(r):
            cb = r * nchunk

            def zrow(i, carry):
                for j in range(nj):
                    acc[i, pl.ds(j * 16, 16)] = jnp.zeros((16,), f32)
                return carry
            lax.fori_loop(0, accr, zrow, 0, unroll=4)

            def issue_small(gg, i):
                pltpu.async_copy(srcp2.at[hsel, cb + gg], ixs[i], sms[i])
                pltpu.async_copy(dlr.at[cb + gg], dls[i], sms[i])
                pltpu.async_copy(gater.at[cb + gg], gts[i], sms[i])

            def wait_small(i):
                pltpu.make_async_copy(srcp2.at[0, 0], ixs[i], sms[i]).wait()
                pltpu.make_async_copy(dlr.at[0], dls[i], sms[i]).wait()
                pltpu.make_async_copy(gater.at[0], gts[i], sms[i]).wait()

            def issue_row(i):
                pltpu.async_copy(y2f.at[ixs[i]], rows_[i], sgs[i])

            def wait_row(i):
                pltpu.make_async_copy(y2f.at[ix0], rows_[i], sgs[i]).wait()

            def accum(i):
                rowb_, gb, db = rows_[i], gts[i], dls[i]

                def one(q, carry):
                    dlv = db[pl.ds(q * 16, 16)]
                    gv = gb[pl.ds(q * 16, 16)]
                    for k in range(16):
                        e1 = q * 16 + k
                        dl = dlv[k]
                        w = gv[k]
                        for j in range(nj):
                            sl = pl.ds(j * 16, 16)
                            acc[dl, sl] = acc[dl, sl] + rowb_[e1, sl] * w
                    return carry
                lax.fori_loop(0, C // 16, one, 0)

            def clamp(v):
                return jnp.minimum(v, nchunk - 1)

            for i in range(4):
                issue_small(i, i)
            for i in range(3):
                wait_small(i)
                issue_row(i)

            def body(qq, carry):
                c = 4 * qq
                wait_small(3)
                issue_row(3)
                wait_row(0)
                accum(0)
                issue_small(clamp(c + 4), 0)
                wait_row(1)
                accum(1)
                issue_small(clamp(c + 5), 1)
                wait_small(0)
                issue_row(0)
                wait_row(2)
                accum(2)
                issue_small(clamp(c + 6), 2)
                wait_small(1)
                issue_row(1)
                wait_row(3)
                accum(3)
                issue_small(clamp(c + 7), 3)
                wait_small(2)
                issue_row(2)
                return carry
            lax.fori_loop(0, nchunk // 4, body, 0)
            wait_row(0)
            wait_row(1)
            wait_row(2)
            wait_small(3)
            if colsplit:
                pltpu.sync_copy(acc.at[pl.ds(0, RSZ)],
                                out.at[h, pl.ds(r * RSZ, RSZ)])
            else:
                pltpu.sync_copy(acc.at[pl.ds(0, RSZ)],
                                out.at[pl.ds(r * RSZ, RSZ)])

        if colsplit:
            one_range(2 * s)
            one_range(2 * s + 1)
        else:
            one_range(h * 16 + s)

    return prop_kernel


def kernel(x, edge_index, edge_attr, params):
    n, d_in = x.shape
    e = edge_index.shape[1]
    npad = NRANGE * RSZ
    assert npad >= n and npad % 128 == 0
    tot = NRANGE * CAP
    totc = tot // C

    src = edge_index[0]
    dst = edge_index[1]

    order = jnp.argsort(dst)
    dst_s = dst[order]
    starts = jnp.searchsorted(
        dst_s, RSZ * jnp.arange(NRANGE, dtype=i32)).astype(i32)
    starts_ext = jnp.concatenate([starts, jnp.full((1,), e, i32)])
    slot = jnp.arange(tot, dtype=i32)
    sb = slot // CAP
    sp = slot % CAP
    gidx = starts_ext[sb] + sp
    valid = gidx < starts_ext[sb + 1]
    gc = jnp.clip(gidx, 0, e - 1)
    order_g = order[gc]
    srcp = jnp.where(valid, src[order_g], 0)
    dl32 = jnp.where(valid, dst_s[gc] - sb * RSZ, RSZ)
    eap = edge_attr[order_g]

    srcp2 = jnp.stack([srcp, srcp + n]).reshape(2, totc, C)
    dl32r = dl32.reshape(totc, C)

    convs = params["convs"]
    norms = params["norms"]
    w1cat = jnp.concatenate([c["W1"] for c in convs], axis=1)
    b1cat = jnp.concatenate([c["b1"] for c in convs])[None]
    w2s = jnp.stack([c["W2"][:, 0] for c in convs])
    eye = jnp.eye(16, dtype=f32)[:NLAYER]
    w2bd = (w2s[:, :, None] * eye[:, None, :]).reshape(288, 16)
    b2cat = jnp.concatenate(
        [jnp.stack([c["b2"][0] for c in convs]), jnp.zeros((7,), f32)])[None]

    rec, rec_t = _edge_mlp(eap, w1cat, b1cat, w2bd, b2cat)

    degp = _build_deg(tot, npad)(rec, dl32r)
    dinv = _dinv(degp, n)

    prop_col = _build_prop(tot, npad, True)
    prop_one = _build_prop(tot, npad, False)

    h = x
    sums = None
    for l in range(NLAYER):
        p = convs[l]
        cout = p["lin"].shape[1]
        colsplit = cout > PH
        gn = norms[l - 1] if l > 0 else None
        y2 = _mm(h, sums, gn, p["lin"], dinv, l)
        gater = rec_t[l].reshape(totc, C)
        if colsplit:
            agg = prop_col(y2.reshape(2 * n, PH), srcp2, dl32r, gater)
        else:
            agg = prop_one(y2, srcp2, dl32r, gater)
        h, sums = _post(agg, y2, dinv, p["bias"][None], l, cout)

    lincat = jnp.concatenate(
        [params["conv_mu"]["lin"], params["conv_logstd"]["lin"]], axis=1)
    bcat = jnp.concatenate(
        [params["conv_mu"]["bias"], params["conv_logstd"]["bias"]])[None]
    y2 = _mm(h, sums, norms[NLAYER - 1], lincat, dinv, NLAYER)
    agg = prop_one(y2, srcp2, dl32r, rec_t[NLAYER].reshape(totc, C))
    mu, logstd = _final(agg, y2, dinv, bcat)
    return (mu, logstd)


# speedup vs baseline: 1.3754x; 1.3754x over previous
import functools

import jax
import jax.numpy as jnp
from jax import lax
from jax.experimental import pallas as pl
from jax.experimental.pallas import tpu as pltpu
from jax.experimental.pallas import tpu_sc as plsc

f32 = jnp.float32
i32 = jnp.int32

C = 128
PH = 128
BN = 2000
BE = 2048
NLAYER = 9
NRANGE = 32
RSZ = 320
CAP = 5888
CAPC = CAP // C


def _emlp_body(ea_ref, w1_ref, b1_ref, w2_ref, b2_ref, rec_ref, rt_ref, *,
               be):
    h = jnp.maximum(
        jnp.dot(ea_ref[...], w1_ref[...], preferred_element_type=f32)
        + b1_ref[...], 0.0)
    logit = jnp.dot(h, w2_ref[...], preferred_element_type=f32) + b2_ref[...]
    s = 1.0 / (1.0 + jnp.exp(-logit))
    col = lax.broadcasted_iota(i32, (be, 16), 1)
    rec = jnp.where(col < NLAYER, s,
                    jnp.where(col == NLAYER, 1.0, 0.0))
    rec_ref[...] = rec
    rt_ref[...] = rec.T


def _edge_mlp(eap, w1cat, b1cat, w2bd, b2cat):
    tot = eap.shape[0]
    grid = tot // BE
    return pl.pallas_call(
        functools.partial(_emlp_body, be=BE),
        grid=(grid,),
        in_specs=[
            pl.BlockSpec((BE, 16), lambda i: (i, 0)),
            pl.BlockSpec((16, 288), lambda i: (0, 0)),
            pl.BlockSpec((1, 288), lambda i: (0, 0)),
            pl.BlockSpec((288, 16), lambda i: (0, 0)),
            pl.BlockSpec((1, 16), lambda i: (0, 0)),
        ],
        out_specs=[
            pl.BlockSpec((BE, 16), lambda i: (i, 0)),
            pl.BlockSpec((16, BE), lambda i: (0, i)),
        ],
        out_shape=[
            jax.ShapeDtypeStruct((tot, 16), f32),
            jax.ShapeDtypeStruct((16, tot), f32),
        ],
    )(eap, w1cat, b1cat, w2bd, b2cat)


def _dinv_body(d_ref, o_ref):
    o_ref[...] = lax.rsqrt(d_ref[...] + 1.0)


def _dinv(degp, n):
    return pl.pallas_call(
        _dinv_body,
        grid=(n // BN,),
        in_specs=[pl.BlockSpec((BN, 16), lambda i: (i, 0))],
        out_specs=pl.BlockSpec((BN, 16), lambda i: (i, 0)),
        out_shape=jax.ShapeDtypeStruct((n, 16), f32),
    )(degp)


def _mm_body(*refs, l, cout, colsplit, gnorm, ntot):
    if gnorm:
        h_ref, sums_ref, gw_ref, gb_ref, gms_ref, lin_ref, dinv_ref, o_ref = refs
        mean = sums_ref[0:1, :] / ntot
        ez2 = sums_ref[1:2, :] / ntot
        ms = gms_ref[...]
        var = ez2 - (2.0 * ms - ms * ms) * mean * mean
        a = gw_ref[...] * lax.rsqrt(var + 1e-5)
        b = gb_ref[...] - a * mean * ms
        u = refs[0][...] * a + b
    else:
        h_ref, lin_ref, dinv_ref, o_ref = refs
        u = h_ref[...]
    xw = jnp.dot(u, lin_ref[...], preferred_element_type=f32)
    y = xw * dinv_ref[...][:, l:l + 1]
    nb = y.shape[0]
    if colsplit:
        r = cout - PH
        o_ref[0] = y[:, :PH]
        o_ref[1, :, :r] = y[:, PH:]
        if r < PH:
            o_ref[1, :, r:] = jnp.zeros((nb, PH - r), f32)
    else:
        o_ref[:, :cout] = y
        if cout < PH:
            o_ref[:, cout:] = jnp.zeros((nb, PH - cout), f32)


def _mm(h, sums, gn, lin, dinv, l):
    n, cin = h.shape
    cout = lin.shape[1]
    colsplit = cout > PH
    gnorm = sums is not None
    body = functools.partial(_mm_body, l=l, cout=cout, colsplit=colsplit,
                             gnorm=gnorm, ntot=float(n))
    in_specs = [pl.BlockSpec((BN, cin), lambda i: (i, 0))]
    args = [h]
    if gnorm:
        in_specs += [
            pl.BlockSpec((2, cin), lambda i: (0, 0)),
            pl.BlockSpec((1, cin), lambda i: (0, 0)),
            pl.BlockSpec((1, cin), lambda i: (0, 0)),
            pl.BlockSpec((1, cin), lambda i: (0, 0)),
        ]
        args += [sums, gn["weight"][None], gn["bias"][None],
                 gn["mean_scale"][None]]
    in_specs += [
        pl.BlockSpec((cin, cout), lambda i: (0, 0)),
        pl.BlockSpec((BN, 16), lambda i: (i, 0)),
    ]
    args += [lin, dinv]
    if colsplit:
        out_specs = pl.BlockSpec((2, BN, PH), lambda i: (0, i, 0))
        out_shape = jax.ShapeDtypeStruct((2, n, PH), f32)
    else:
        out_specs = pl.BlockSpec((BN, PH), lambda i: (i, 0))
        out_shape = jax.ShapeDtypeStruct((n, PH), f32)
    return pl.pallas_call(
        body,
        grid=(n // BN,),
        in_specs=in_specs,
        out_specs=out_specs,
        out_shape=out_shape,
    )(*args)


def _post_body(agg_ref, y_ref, dinv_ref, bias_ref, h_ref, sums_ref, *,
               l, cout, colsplit, relu):
    i = pl.program_id(0)
    if colsplit:
        a0 = agg_ref[0] + y_ref[0]
        a1 = agg_ref[1] + y_ref[1]
        z = jnp.concatenate([a0, a1[:, :cout - PH]], axis=1)
    else:
        z = (agg_ref[...] + y_ref[...])[:, :cout]
    z = z * dinv_ref[...][:, l:l + 1] + bias_ref[...]
    if relu:
        z = jnp.maximum(z, 0.0)
    h_ref[...] = z
    ps = jnp.concatenate([jnp.sum(z, axis=0, keepdims=True),
                          jnp.sum(z * z, axis=0, keepdims=True)], axis=0)

    @pl.when(i == 0)
    def _():
        sums_ref[...] = ps

    @pl.when(i > 0)
    def _():
        sums_ref[...] += ps


def _post(agg, y, dinv, bias, l, cout):
    colsplit = y.ndim == 3
    n = y.shape[1] if colsplit else y.shape[0]
    body = functools.partial(_post_body, l=l, cout=cout, colsplit=colsplit,
                             relu=True)
    if colsplit:
        agg_spec = pl.BlockSpec((2, BN, PH), lambda i: (0, i, 0))
        y_spec = pl.BlockSpec((2, BN, PH), lambda i: (0, i, 0))
    else:
        agg_spec = pl.BlockSpec((BN, PH), lambda i: (i, 0))
        y_spec = pl.BlockSpec((BN, PH), lambda i: (i, 0))
    return pl.pallas_call(
        body,
        grid=(n // BN,),
        in_specs=[
            agg_spec,
            y_spec,
            pl.BlockSpec((BN, 16), lambda i: (i, 0)),
            pl.BlockSpec((1, cout), lambda i: (0, 0)),
        ],
        out_specs=[
            pl.BlockSpec((BN, cout), lambda i: (i, 0)),
            pl.BlockSpec((2, cout), lambda i: (0, 0)),
        ],
        out_shape=[
            jax.ShapeDtypeStruct((n, cout), f32),
            jax.ShapeDtypeStruct((2, cout), f32),
        ],
    )(agg, y, dinv, bias)


def _final_body(agg_ref, y_ref, dinv_ref, bias_ref, mu_ref, ls_ref, *, l):
    z = agg_ref[...] + y_ref[...]
    z = z * dinv_ref[...][:, l:l + 1] + bias_ref[...]
    mu_ref[...] = z[:, :64]
    ls_ref[...] = z[:, 64:]


def _final(agg, y, dinv, bias):
    n = y.shape[0]
    body = functools.partial(_final_body, l=NLAYER)
    return pl.pallas_call(
        body,
        grid=(n // BN,),
        in_specs=[
            pl.BlockSpec((BN, PH), lambda i: (i, 0)),
            pl.BlockSpec((BN, PH), lambda i: (i, 0)),
            pl.BlockSpec((BN, 16), lambda i: (i, 0)),
            pl.BlockSpec((1, 128), lambda i: (0, 0)),
        ],
        out_specs=[
            pl.BlockSpec((BN, 64), lambda i: (i, 0)),
            pl.BlockSpec((BN, 64), lambda i: (i, 0)),
        ],
        out_shape=[
            jax.ShapeDtypeStruct((n, 64), f32),
            jax.ShapeDtypeStruct((n, 64), f32),
        ],
    )(agg, y, dinv, bias)


_MESH = dict(core_axis_name="c", subcore_axis_name="s")


def _build_deg(tot, npad):
    totc = tot // C
    accr = RSZ + 8
    mesh = plsc.VectorSubcoreMesh(**_MESH)

    @functools.partial(
        pl.kernel, mesh=mesh,
        out_type=jax.ShapeDtypeStruct((npad, 16), f32),
        scratch_types=[
            pltpu.VMEM((C, 16), f32),
            pltpu.VMEM((C, 16), f32),
            pltpu.VMEM((C,), i32),
            pltpu.VMEM((C,), i32),
            pltpu.SemaphoreType.DMA,
            pltpu.SemaphoreType.DMA,
            pltpu.SemaphoreType.DMA,
            pltpu.SemaphoreType.DMA,
            pltpu.VMEM((accr, 16), f32),
        ])
    def deg_kernel(rec, dlr, out, reca, recb, dla, dlb, sra, srb, sda, sdb,
                   acc):
        h = lax.axis_index("c")
        s = lax.axis_index("s")
        r = h * 16 + s
        cb = r * CAPC

        def zrow(i, carry):
            acc[i, :] = jnp.zeros((16,), f32)
            return carry
        lax.fori_loop(0, accr, zrow, 0, unroll=4)

        def rows(gg):
            return pl.ds(pl.multiple_of((cb + gg) * C, C), C)

        def issue(gg, rb, db, sr, sd):
            pltpu.async_copy(rec.at[rows(gg)], rb, sr)
            pltpu.async_copy(dlr.at[cb + gg], db, sd)

        def wait(rb, db, sr, sd):
            pltpu.make_async_copy(rec.at[rows(0)], rb, sr).wait()
            pltpu.make_async_copy(dlr.at[0], db, sd).wait()

        def accum(rb, db):
            def one(q, carry):
                dlv = db[pl.ds(q * 16, 16)]
                for k in range(16):
                    e1 = q * 16 + k
                    dl = dlv[k]
                    acc[dl, :] = acc[dl, :] + rb[e1, :]
                return carry
            lax.fori_loop(0, C // 16, one, 0)

        issue(0, reca, dla, sra, sda)
        issue(1, recb, dlb, srb, sdb)

        def body(gp, carry):
            g0 = 2 * gp
            wait(reca, dla, sra, sda)
            accum(reca, dla)
            issue(jnp.minimum(g0 + 2, CAPC - 1), reca, dla, sra, sda)
            wait(recb, dlb, srb, sdb)
            accum(recb, dlb)
            issue(jnp.minimum(g0 + 3, CAPC - 1), recb, dlb, srb, sdb)
            return carry
        lax.fori_loop(0, CAPC // 2, body, 0)
        wait(reca, dla, sra, sda)
        wait(recb, dlb, srb, sdb)
        pltpu.sync_copy(acc.at[pl.ds(0, RSZ)], out.at[pl.ds(r * RSZ, RSZ)])

    return deg_kernel


def _build_prop(tot, npad, colsplit):
    nchunk = CAPC
    accr = RSZ + 8
    nj = PH // 16
    mesh = plsc.VectorSubcoreMesh(**_MESH)

    @functools.partial(
        pl.kernel, mesh=mesh,
        out_type=(jax.ShapeDtypeStruct((2, npad, PH), f32) if colsplit
                  else jax.ShapeDtypeStruct((npad, PH), f32)),
        scratch_types=[
            pltpu.VMEM((C, PH), f32),
            pltpu.VMEM((C, PH), f32),
            pltpu.VMEM((C,), f32),
            pltpu.VMEM((C,), f32),
            pltpu.VMEM((C,), i32),
            pltpu.VMEM((C,), i32),
            pltpu.VMEM((C,), i32),
            pltpu.VMEM((C,), i32),
            pltpu.SemaphoreType.DMA,
            pltpu.SemaphoreType.DMA,
            pltpu.SemaphoreType.DMA,
            pltpu.SemaphoreType.DMA,
            pltpu.VMEM((accr, PH), f32),
        ])
    def prop_kernel(y2f, srcp2, dlr, gater, out, rowa, rowb, gta, gtb,
                    idxa, idxb, dla, dlb, sga, sgb, sma, smb, acc):
        h = lax.axis_index("c")
        s = lax.axis_index("s")
        hsel = h if colsplit else 0

        def one_range(r):
            cb = r * nchunk

            def zrow(i, carry):
                for j in range(nj):
                    acc[i, pl.ds(j * 16, 16)] = jnp.zeros((16,), f32)
                return carry
            lax.fori_loop(0, accr, zrow, 0, unroll=4)

            def issue_small(gg, ib, db, gb, sm):
                pltpu.async_copy(srcp2.at[hsel, cb + gg], ib, sm)
                pltpu.async_copy(dlr.at[cb + gg], db, sm)
                pltpu.async_copy(gater.at[cb + gg], gb, sm)

            def wait_small(ib, db, gb, sm):
                pltpu.make_async_copy(srcp2.at[0, 0], ib, sm).wait()
                pltpu.make_async_copy(dlr.at[0], db, sm).wait()
                pltpu.make_async_copy(gater.at[0], gb, sm).wait()

            def issue_row(ib, rb, sg):
                pltpu.async_copy(y2f.at[ib], rb, sg)

            def wait_row(rb, sg):
                pltpu.make_async_copy(y2f.at[idxa], rb, sg).wait()

            def accum(rowb_, gb, db):
                def one(q, carry):
                    dlv = db[pl.ds(q * 16, 16)]
                    gv = gb[pl.ds(q * 16, 16)]
                    for k in range(16):
                        e1 = q * 16 + k
                        dl = dlv[k]
                        w = gv[k]
                        for j in range(nj):
                            sl = pl.ds(j * 16, 16)
                            acc[dl, sl] = acc[dl, sl] + rowb_[e1, sl] * w
                    return carry
                lax.fori_loop(0, C // 16, one, 0)

            issue_small(0, idxa, dla, gta, sma)
            wait_small(idxa, dla, gta, sma)
            issue_row(idxa, rowa, sga)
            issue_small(1, idxb, dlb, gtb, smb)

            def body(gp, carry):
                g0 = 2 * gp
                wait_small(idxb, dlb, gtb, smb)
                issue_row(idxb, rowb, sgb)
                wait_row(rowa, sga)
                accum(rowa, gta, dla)
                issue_small(jnp.minimum(g0 + 2, nchunk - 1),
                            idxa, dla, gta, sma)
                wait_small(idxa, dla, gta, sma)
                issue_row(idxa, rowa, sga)
                wait_row(rowb, sgb)
                accum(rowb, gtb, dlb)
                issue_small(jnp.minimum(g0 + 3, nchunk - 1),
                            idxb, dlb, gtb, smb)
                return carry
            lax.fori_loop(0, nchunk // 2, body, 0)
            wait_row(rowa, sga)
            wait_small(idxb, dlb, gtb, smb)
            if colsplit:
                pltpu.sync_copy(acc.at[pl.ds(0, RSZ)],
                                out.at[h, pl.ds(r * RSZ, RSZ)])
            else:
                pltpu.sync_copy(acc.at[pl.ds(0, RSZ)],
                                out.at[pl.ds(r * RSZ, RSZ)])

        if colsplit:
            one_range(2 * s)
            one_range(2 * s + 1)
        else:
            one_range(h * 16 + s)

    return prop_kernel


def kernel(x, edge_index, edge_attr, params):
    n, d_in = x.shape
    e = edge_index.shape[1]
    npad = NRANGE * RSZ
    assert npad >= n and npad % 128 == 0
    tot = NRANGE * CAP
    totc = tot // C

    src = edge_index[0]
    dst = edge_index[1]

    order = jnp.argsort(dst)
    dst_s = dst[order]
    starts = jnp.searchsorted(
        dst_s, RSZ * jnp.arange(NRANGE, dtype=i32)).astype(i32)
    starts_ext = jnp.concatenate([starts, jnp.full((1,), e, i32)])
    slot = jnp.arange(tot, dtype=i32)
    sb = slot // CAP
    sp = slot % CAP
    gidx = starts_ext[sb] + sp
    valid = gidx < starts_ext[sb + 1]
    gc = jnp.clip(gidx, 0, e - 1)
    order_g = order[gc]
    srcp = jnp.where(valid, src[order_g], 0)
    dl32 = jnp.where(valid, dst_s[gc] - sb * RSZ, RSZ)
    eap = edge_attr[order_g]

    srcp2 = jnp.stack([srcp, srcp + n]).reshape(2, totc, C)
    dl32r = dl32.reshape(totc, C)

    convs = params["convs"]
    norms = params["norms"]
    w1cat = jnp.concatenate([c["W1"] for c in convs], axis=1)
    b1cat = jnp.concatenate([c["b1"] for c in convs])[None]
    w2s = jnp.stack([c["W2"][:, 0] for c in convs])
    eye = jnp.eye(16, dtype=f32)[:NLAYER]
    w2bd = (w2s[:, :, None] * eye[:, None, :]).reshape(288, 16)
    b2cat = jnp.concatenate(
        [jnp.stack([c["b2"][0] for c in convs]), jnp.zeros((7,), f32)])[None]

    rec, rec_t = _edge_mlp(eap, w1cat, b1cat, w2bd, b2cat)

    degp = _build_deg(tot, npad)(rec, dl32r)
    dinv = _dinv(degp, n)

    prop_col = _build_prop(tot, npad, True)
    prop_one = _build_prop(tot, npad, False)

    h = x
    sums = None
    for l in range(NLAYER):
        p = convs[l]
        cout = p["lin"].shape[1]
        colsplit = cout > PH
        gn = norms[l - 1] if l > 0 else None
        y2 = _mm(h, sums, gn, p["lin"], dinv, l)
        gater = rec_t[l].reshape(totc, C)
        if colsplit:
            agg = prop_col(y2.reshape(2 * n, PH), srcp2, dl32r, gater)
        else:
            agg = prop_one(y2, srcp2, dl32r, gater)
        h, sums = _post(agg, y2, dinv, p["bias"][None], l, cout)

    lincat = jnp.concatenate(
        [params["conv_mu"]["lin"], params["conv_logstd"]["lin"]], axis=1)
    bcat = jnp.concatenate(
        [params["conv_mu"]["bias"], params["conv_logstd"]["bias"]])[None]
    y2 = _mm(h, sums, norms[NLAYER - 1], lincat, dinv, NLAYER)
    agg = prop_one(y2, srcp2, dl32r, rec_t[NLAYER].reshape(totc, C))
    mu, logstd = _final(agg, y2, dinv, bcat)
    return (mu, logstd)


# speedup vs baseline: 1.6969x; 1.2338x over previous
import functools

import jax
import jax.numpy as jnp
from jax import lax
from jax.experimental import pallas as pl
from jax.experimental.pallas import tpu as pltpu
from jax.experimental.pallas import tpu_sc as plsc

f32 = jnp.float32
i32 = jnp.int32

C = 128
PH = 128
BN = 2000
BE = 2048
NLAYER = 9
NRANGE = 32
RSZ = 320
CAP = 5632
CAPC = CAP // C


def _emlp_body(ea_ref, w1_ref, b1_ref, w2_ref, b2_ref, rec_ref, rt_ref, *,
               be):
    h = jnp.maximum(
        jnp.dot(ea_ref[...], w1_ref[...], preferred_element_type=f32)
        + b1_ref[...], 0.0)
    logit = jnp.dot(h, w2_ref[...], preferred_element_type=f32) + b2_ref[...]
    s = 1.0 / (1.0 + jnp.exp(-logit))
    col = lax.broadcasted_iota(i32, (be, 16), 1)
    rec = jnp.where(col < NLAYER, s,
                    jnp.where(col == NLAYER, 1.0, 0.0))
    rec_ref[...] = rec
    rt_ref[...] = rec.T


def _edge_mlp(eap, w1cat, b1cat, w2bd, b2cat):
    tot = eap.shape[0]
    grid = tot // BE
    return pl.pallas_call(
        functools.partial(_emlp_body, be=BE),
        grid=(grid,),
        in_specs=[
            pl.BlockSpec((BE, 16), lambda i: (i, 0)),
            pl.BlockSpec((16, 288), lambda i: (0, 0)),
            pl.BlockSpec((1, 288), lambda i: (0, 0)),
            pl.BlockSpec((288, 16), lambda i: (0, 0)),
            pl.BlockSpec((1, 16), lambda i: (0, 0)),
        ],
        out_specs=[
            pl.BlockSpec((BE, 16), lambda i: (i, 0)),
            pl.BlockSpec((16, BE), lambda i: (0, i)),
        ],
        out_shape=[
            jax.ShapeDtypeStruct((tot, 16), f32),
            jax.ShapeDtypeStruct((16, tot), f32),
        ],
    )(eap, w1cat, b1cat, w2bd, b2cat)


def _dinv_body(d_ref, o_ref):
    o_ref[...] = lax.rsqrt(d_ref[...] + 1.0)


def _dinv(degp, n):
    return pl.pallas_call(
        _dinv_body,
        grid=(n // BN,),
        in_specs=[pl.BlockSpec((BN, 16), lambda i: (i, 0))],
        out_specs=pl.BlockSpec((BN, 16), lambda i: (i, 0)),
        out_shape=jax.ShapeDtypeStruct((n, 16), f32),
    )(degp)


def _mm_body(*refs, l, cout, colsplit, gnorm, ntot):
    if gnorm:
        h_ref, sums_ref, gw_ref, gb_ref, gms_ref, lin_ref, dinv_ref, o_ref = refs
        mean = sums_ref[0:1, :] / ntot
        ez2 = sums_ref[1:2, :] / ntot
        ms = gms_ref[...]
        var = ez2 - (2.0 * ms - ms * ms) * mean * mean
        a = gw_ref[...] * lax.rsqrt(var + 1e-5)
        b = gb_ref[...] - a * mean * ms
        u = refs[0][...] * a + b
    else:
        h_ref, lin_ref, dinv_ref, o_ref = refs
        u = h_ref[...]
    xw = jnp.dot(u, lin_ref[...], preferred_element_type=f32)
    y = xw * dinv_ref[...][:, l:l + 1]
    nb = y.shape[0]
    if colsplit:
        r = cout - PH
        o_ref[0] = y[:, :PH]
        o_ref[1, :, :r] = y[:, PH:]
        if r < PH:
            o_ref[1, :, r:] = jnp.zeros((nb, PH - r), f32)
    else:
        o_ref[:, :cout] = y
        if cout < PH:
            o_ref[:, cout:] = jnp.zeros((nb, PH - cout), f32)


def _mm(h, sums, gn, lin, dinv, l):
    n, cin = h.shape
    cout = lin.shape[1]
    colsplit = cout > PH
    gnorm = sums is not None
    body = functools.partial(_mm_body, l=l, cout=cout, colsplit=colsplit,
                             gnorm=gnorm, ntot=float(n))
    in_specs = [pl.BlockSpec((BN, cin), lambda i: (i, 0))]
    args = [h]
    if gnorm:
        in_specs += [
            pl.BlockSpec((2, cin), lambda i: (0, 0)),
            pl.BlockSpec((1, cin), lambda i: (0, 0)),
            pl.BlockSpec((1, cin), lambda i: (0, 0)),
            pl.BlockSpec((1, cin), lambda i: (0, 0)),
        ]
        args += [sums, gn["weight"][None], gn["bias"][None],
                 gn["mean_scale"][None]]
    in_specs += [
        pl.BlockSpec((cin, cout), lambda i: (0, 0)),
        pl.BlockSpec((BN, 16), lambda i: (i, 0)),
    ]
    args += [lin, dinv]
    if colsplit:
        out_specs = pl.BlockSpec((2, BN, PH), lambda i: (0, i, 0))
        out_shape = jax.ShapeDtypeStruct((2, n, PH), f32)
    else:
        out_specs = pl.BlockSpec((BN, PH), lambda i: (i, 0))
        out_shape = jax.ShapeDtypeStruct((n, PH), f32)
    return pl.pallas_call(
        body,
        grid=(n // BN,),
        in_specs=in_specs,
        out_specs=out_specs,
        out_shape=out_shape,
    )(*args)


def _post_body(agg_ref, y_ref, dinv_ref, bias_ref, h_ref, sums_ref, *,
               l, cout, colsplit, relu):
    i = pl.program_id(0)
    if colsplit:
        a0 = agg_ref[0] + y_ref[0]
        a1 = agg_ref[1] + y_ref[1]
        z = jnp.concatenate([a0, a1[:, :cout - PH]], axis=1)
    else:
        z = (agg_ref[...] + y_ref[...])[:, :cout]
    z = z * dinv_ref[...][:, l:l + 1] + bias_ref[...]
    if relu:
        z = jnp.maximum(z, 0.0)
    h_ref[...] = z
    ps = jnp.concatenate([jnp.sum(z, axis=0, keepdims=True),
                          jnp.sum(z * z, axis=0, keepdims=True)], axis=0)

    @pl.when(i == 0)
    def _():
        sums_ref[...] = ps

    @pl.when(i > 0)
    def _():
        sums_ref[...] += ps


def _post(agg, y, dinv, bias, l, cout):
    colsplit = y.ndim == 3
    n = y.shape[1] if colsplit else y.shape[0]
    body = functools.partial(_post_body, l=l, cout=cout, colsplit=colsplit,
                             relu=True)
    if colsplit:
        agg_spec = pl.BlockSpec((2, BN, PH), lambda i: (0, i, 0))
        y_spec = pl.BlockSpec((2, BN, PH), lambda i: (0, i, 0))
    else:
        agg_spec = pl.BlockSpec((BN, PH), lambda i: (i, 0))
        y_spec = pl.BlockSpec((BN, PH), lambda i: (i, 0))
    return pl.pallas_call(
        body,
        grid=(n // BN,),
        in_specs=[
            agg_spec,
            y_spec,
            pl.BlockSpec((BN, 16), lambda i: (i, 0)),
            pl.BlockSpec((1, cout), lambda i: (0, 0)),
        ],
        out_specs=[
            pl.BlockSpec((BN, cout), lambda i: (i, 0)),
            pl.BlockSpec((2, cout), lambda i: (0, 0)),
        ],
        out_shape=[
            jax.ShapeDtypeStruct((n, cout), f32),
            jax.ShapeDtypeStruct((2, cout), f32),
        ],
    )(agg, y, dinv, bias)


def _final_body(agg_ref, y_ref, dinv_ref, bias_ref, mu_ref, ls_ref, *, l):
    z = agg_ref[...] + y_ref[...]
    z = z * dinv_ref[...][:, l:l + 1] + bias_ref[...]
    mu_ref[...] = z[:, :64]
    ls_ref[...] = z[:, 64:]


def _final(agg, y, dinv, bias):
    n = y.shape[0]
    body = functools.partial(_final_body, l=NLAYER)
    return pl.pallas_call(
        body,
        grid=(n // BN,),
        in_specs=[
            pl.BlockSpec((BN, PH), lambda i: (i, 0)),
            pl.BlockSpec((BN, PH), lambda i: (i, 0)),
            pl.BlockSpec((BN, 16), lambda i: (i, 0)),
            pl.BlockSpec((1, 128), lambda i: (0, 0)),
        ],
        out_specs=[
            pl.BlockSpec((BN, 64), lambda i: (i, 0)),
            pl.BlockSpec((BN, 64), lambda i: (i, 0)),
        ],
        out_shape=[
            jax.ShapeDtypeStruct((n, 64), f32),
            jax.ShapeDtypeStruct((n, 64), f32),
        ],
    )(agg, y, dinv, bias)


_MESH = dict(core_axis_name="c", subcore_axis_name="s")


def _build_deg(tot, npad):
    totc = tot // C
    accr = RSZ + 8
    mesh = plsc.VectorSubcoreMesh(**_MESH)

    @functools.partial(
        pl.kernel, mesh=mesh,
        out_type=jax.ShapeDtypeStruct((npad, 16), f32),
        scratch_types=[
            pltpu.VMEM((C, 16), f32),
            pltpu.VMEM((C, 16), f32),
            pltpu.VMEM((C,), i32),
            pltpu.VMEM((C,), i32),
            pltpu.SemaphoreType.DMA,
            pltpu.SemaphoreType.DMA,
            pltpu.SemaphoreType.DMA,
            pltpu.SemaphoreType.DMA,
            pltpu.VMEM((accr, 16), f32),
        ])
    def deg_kernel(rec, dlr, out, reca, recb, dla, dlb, sra, srb, sda, sdb,
                   acc):
        h = lax.axis_index("c")
        s = lax.axis_index("s")
        r = h * 16 + s
        cb = r * CAPC

        def zrow(i, carry):
            acc[i, :] = jnp.zeros((16,), f32)
            return carry
        lax.fori_loop(0, accr, zrow, 0, unroll=4)

        def rows(gg):
            return pl.ds(pl.multiple_of((cb + gg) * C, C), C)

        def issue(gg, rb, db, sr, sd):
            pltpu.async_copy(rec.at[rows(gg)], rb, sr)
            pltpu.async_copy(dlr.at[cb + gg], db, sd)

        def wait(rb, db, sr, sd):
            pltpu.make_async_copy(rec.at[rows(0)], rb, sr).wait()
            pltpu.make_async_copy(dlr.at[0], db, sd).wait()

        def accum(rb, db):
            def one(q, carry):
                dlv = db[pl.ds(q * 16, 16)]
                for k in range(16):
                    e1 = q * 16 + k
                    dl = dlv[k]
                    acc[dl, :] = acc[dl, :] + rb[e1, :]
                return carry
            lax.fori_loop(0, C // 16, one, 0)

        issue(0, reca, dla, sra, sda)
        issue(1, recb, dlb, srb, sdb)

        def body(gp, carry):
            g0 = 2 * gp
            wait(reca, dla, sra, sda)
            accum(reca, dla)
            issue(jnp.minimum(g0 + 2, CAPC - 1), reca, dla, sra, sda)
            wait(recb, dlb, srb, sdb)
            accum(recb, dlb)
            issue(jnp.minimum(g0 + 3, CAPC - 1), recb, dlb, srb, sdb)
            return carry
        lax.fori_loop(0, CAPC // 2, body, 0)
        wait(reca, dla, sra, sda)
        wait(recb, dlb, srb, sdb)
        pltpu.sync_copy(acc.at[pl.ds(0, RSZ)], out.at[pl.ds(r * RSZ, RSZ)])

    return deg_kernel


def _build_prop(tot, npad, colsplit):
    nchunk = CAPC
    accr = RSZ + 8
    nj = PH // 16
    mesh = plsc.VectorSubcoreMesh(**_MESH)

    @functools.partial(
        pl.kernel, mesh=mesh,
        out_type=(jax.ShapeDtypeStruct((2, npad, PH), f32) if colsplit
                  else jax.ShapeDtypeStruct((npad, PH), f32)),
        scratch_types=[
            pltpu.VMEM((C, PH), f32),
            pltpu.VMEM((C, PH), f32),
            pltpu.VMEM((C,), f32),
            pltpu.VMEM((C,), f32),
            pltpu.VMEM((C,), i32),
            pltpu.VMEM((C,), i32),
            pltpu.VMEM((C,), i32),
            pltpu.VMEM((C,), i32),
            pltpu.SemaphoreType.DMA,
            pltpu.SemaphoreType.DMA,
            pltpu.SemaphoreType.DMA,
            pltpu.SemaphoreType.DMA,
            pltpu.VMEM((accr, PH), f32),
        ])
    def prop_kernel(y2f, srcp2, dlr, gater, out, rowa, rowb, gta, gtb,
                    idxa, idxb, dla, dlb, sga, sgb, sma, smb, acc):
        h = lax.axis_index("c")
        s = lax.axis_index("s")
        hsel = h if colsplit else 0

        def one_range(r):
            cb = r * nchunk

            def zrow(i, carry):
                for j in range(nj):
                    acc[i, pl.ds(j * 16, 16)] = jnp.zeros((16,), f32)
                return carry
            lax.fori_loop(0, accr, zrow, 0, unroll=4)

            def issue_small(gg, ib, db, gb, sm):
                pltpu.async_copy(srcp2.at[hsel, cb + gg], ib, sm)
                pltpu.async_copy(dlr.at[cb + gg], db, sm)
                pltpu.async_copy(gater.at[cb + gg], gb, sm)

            def wait_small(ib, db, gb, sm):
                pltpu.make_async_copy(srcp2.at[0, 0], ib, sm).wait()
                pltpu.make_async_copy(dlr.at[0], db, sm).wait()
                pltpu.make_async_copy(gater.at[0], gb, sm).wait()

            def issue_row(ib, rb, sg):
                pltpu.async_copy(y2f.at[ib], rb, sg)

            def wait_row(rb, sg):
                pltpu.make_async_copy(y2f.at[idxa], rb, sg).wait()

            def accum(rowb_, gb, db):
                def one(q, carry):
                    dlv = db[pl.ds(q * 16, 16)]
                    gv = gb[pl.ds(q * 16, 16)]
                    for k in range(16):
                        e1 = q * 16 + k
                        dl = dlv[k]
                        w = gv[k]
                        for j in range(nj):
                            sl = pl.ds(j * 16, 16)
                            acc[dl, sl] = acc[dl, sl] + rowb_[e1, sl] * w
                    return carry
                lax.fori_loop(0, C // 16, one, 0)

            issue_small(0, idxa, dla, gta, sma)
            wait_small(idxa, dla, gta, sma)
            issue_row(idxa, rowa, sga)
            issue_small(1, idxb, dlb, gtb, smb)

            def body(gp, carry):
                g0 = 2 * gp
                wait_small(idxb, dlb, gtb, smb)
                issue_row(idxb, rowb, sgb)
                wait_row(rowa, sga)
                accum(rowa, gta, dla)
                issue_small(jnp.minimum(g0 + 2, nchunk - 1),
                            idxa, dla, gta, sma)
                wait_small(idxa, dla, gta, sma)
                issue_row(idxa, rowa, sga)
                wait_row(rowb, sgb)
                accum(rowb, gtb, dlb)
                issue_small(jnp.minimum(g0 + 3, nchunk - 1),
                            idxb, dlb, gtb, smb)
                return carry
            lax.fori_loop(0, nchunk // 2, body, 0)
            wait_row(rowa, sga)
            wait_small(idxb, dlb, gtb, smb)
            if colsplit:
                pltpu.sync_copy(acc.at[pl.ds(0, RSZ)],
                                out.at[h, pl.ds(r * RSZ, RSZ)])
            else:
                pltpu.sync_copy(acc.at[pl.ds(0, RSZ)],
                                out.at[pl.ds(r * RSZ, RSZ)])

        if colsplit:
            one_range(2 * s)
            one_range(2 * s + 1)
        else:
            one_range(h * 16 + s)

    return prop_kernel


def kernel(x, edge_index, edge_attr, params):
    n, d_in = x.shape
    e = edge_index.shape[1]
    npad = NRANGE * RSZ
    assert npad >= n and npad % 128 == 0
    tot = NRANGE * CAP
    totc = tot // C

    src = edge_index[0]
    dst = edge_index[1]

    order = jnp.argsort(dst)
    dst_s = dst[order]
    starts = jnp.searchsorted(
        dst_s, RSZ * jnp.arange(NRANGE, dtype=i32)).astype(i32)
    starts_ext = jnp.concatenate([starts, jnp.full((1,), e, i32)])
    slot = jnp.arange(tot, dtype=i32)
    sb = slot // CAP
    sp = slot % CAP
    gidx = starts_ext[sb] + sp
    valid = gidx < starts_ext[sb + 1]
    gc = jnp.clip(gidx, 0, e - 1)
    order_g = order[gc]
    srcp = jnp.where(valid, src[order_g], 0)
    dl32 = jnp.where(valid, dst_s[gc] - sb * RSZ, RSZ)
    eap = edge_attr[order_g]

    srcp2 = jnp.stack([srcp, srcp + n]).reshape(2, totc, C)
    dl32r = dl32.reshape(totc, C)

    convs = params["convs"]
    norms = params["norms"]
    w1cat = jnp.concatenate([c["W1"] for c in convs], axis=1)
    b1cat = jnp.concatenate([c["b1"] for c in convs])[None]
    w2s = jnp.stack([c["W2"][:, 0] for c in convs])
    eye = jnp.eye(16, dtype=f32)[:NLAYER]
    w2bd = (w2s[:, :, None] * eye[:, None, :]).reshape(288, 16)
    b2cat = jnp.concatenate(
        [jnp.stack([c["b2"][0] for c in convs]), jnp.zeros((7,), f32)])[None]

    rec, rec_t = _edge_mlp(eap, w1cat, b1cat, w2bd, b2cat)

    degp = _build_deg(tot, npad)(rec, dl32r)
    dinv = _dinv(degp, n)

    prop_col = _build_prop(tot, npad, True)
    prop_one = _build_prop(tot, npad, False)

    h = x
    sums = None
    for l in range(NLAYER):
        p = convs[l]
        cout = p["lin"].shape[1]
        colsplit = cout > PH
        gn = norms[l - 1] if l > 0 else None
        y2 = _mm(h, sums, gn, p["lin"], dinv, l)
        gater = rec_t[l].reshape(totc, C)
        if colsplit:
            agg = prop_col(y2.reshape(2 * n, PH), srcp2, dl32r, gater)
        else:
            agg = prop_one(y2, srcp2, dl32r, gater)
        h, sums = _post(agg, y2, dinv, p["bias"][None], l, cout)

    lincat = jnp.concatenate(
        [params["conv_mu"]["lin"], params["conv_logstd"]["lin"]], axis=1)
    bcat = jnp.concatenate(
        [params["conv_mu"]["bias"], params["conv_logstd"]["bias"]])[None]
    y2 = _mm(h, sums, norms[NLAYER - 1], lincat, dinv, NLAYER)
    agg = prop_one(y2, srcp2, dl32r, rec_t[NLAYER].reshape(totc, C))
    mu, logstd = _final(agg, y2, dinv, bcat)
    return (mu, logstd)
